# baseline (speedup 1.0000x reference)
import jax
import jax.numpy as jnp
from jax import lax
from jax.experimental import pallas as pl
from jax.experimental.pallas import tpu as pltpu

_N = 128
_R = 16
_NB = _N // _R
_NCLUST = 8
_F32 = jnp.float32
_I32 = jnp.int32

_HI = lax.Precision.HIGHEST


def _mm(a, b, precision=None):
    return lax.dot_general(a, b, dimension_numbers=(((1,), (0,)), ((), ())),
                           preferred_element_type=_F32, precision=precision)


def _dense_kernel(emb_ref, px_ref, py_ref, pxb_ref, pyb_ref,
                  W1aT_ref, W1bT_ref, wd_ref, g_ref,
                  W2T_ref, b2_ref, W3T_ref, b3_ref,
                  We1T_ref, be1_ref, We2T_ref, be2_ref,
                  Wp1T_ref, bp1_ref, Wp2T_ref, bp2_ref,
                  pred_ref, eff_ref, pri_ref, A_ref, B_ref):
    b = pl.program_id(0)

    @pl.when(b == 0)
    def _():
        A_ref[...] = _mm(emb_ref[...], W1aT_ref[...]) + g_ref[...]
        B_ref[...] = _mm(emb_ref[...], W1bT_ref[...])
        hp = jnp.maximum(_mm(emb_ref[...], Wp1T_ref[...]) + bp1_ref[...], 0.0)
        pri_ref[...] = jax.nn.sigmoid(_mm(hp, Wp2T_ref[...]) + bp2_ref[...])

    pxi = pxb_ref[...].reshape(_R, 1, 1)
    pyi = pyb_ref[...].reshape(_R, 1, 1)
    pxj = px_ref[...].reshape(1, _N, 1)
    pyj = py_ref[...].reshape(1, _N, 1)
    dx = jnp.broadcast_to(pxi, (_R, _N, 1)) - jnp.broadcast_to(pxj, (_R, _N, 1))
    dy = jnp.broadcast_to(pyi, (_R, _N, 1)) - jnp.broadcast_to(pyj, (_R, _N, 1))
    dist = jnp.sqrt(dx * dx + dy * dy).reshape(_R * _N, 1)

    he = jnp.maximum(_mm(dist * (1.0 / 1000.0), We1T_ref[...]) + be1_ref[...],
                     0.0)
    se = jax.nn.sigmoid(_mm(he, We2T_ref[...]) + be2_ref[...])
    eff_ref[...] = (0.85 + 0.13 * se).reshape(_R, _N, 1)

    a3 = A_ref[pl.ds(b * _R, _R), :].reshape(_R, 1, _N)
    ab = jnp.broadcast_to(a3, (_R, _N, _N)).reshape(_R * _N, _N)
    bb = jnp.broadcast_to(B_ref[...].reshape(1, _N, _N),
                          (_R, _N, _N)).reshape(_R * _N, _N)
    h1 = jnp.maximum(ab + bb + dist * wd_ref[...], 0.0)
    h2 = jnp.maximum(_mm(h1, W2T_ref[...]) + b2_ref[...], 0.0)
    pr = _mm(h2, W3T_ref[...]) + b3_ref[...]
    pred_ref[...] = jax.nn.softplus(pr).reshape(_R, _N, 1)


def _run_dense(emb, px, py, W1aT, W1bT, wd_row, g_row, W2T, b2r,
               W3T, b3r, We1T, be1r, We2T, be2r, Wp1T, bp1r, Wp2T, bp2r):
    full = lambda shp: pl.BlockSpec(shp, lambda b: tuple(0 for _ in shp))
    in_specs = [
        full((_N, _N)),
        full((_N, 1)), full((_N, 1)),
        pl.BlockSpec((_R, 1), lambda b: (b, 0)),
        pl.BlockSpec((_R, 1), lambda b: (b, 0)),
        full(W1aT.shape), full(W1bT.shape), full(wd_row.shape),
        full(g_row.shape), full(W2T.shape), full(b2r.shape),
        full(W3T.shape), full(b3r.shape), full(We1T.shape), full(be1r.shape),
        full(We2T.shape), full(be2r.shape), full(Wp1T.shape), full(bp1r.shape),
        full(Wp2T.shape), full(bp2r.shape),
    ]
    out_specs = [
        pl.BlockSpec((_R, _N, 1), lambda b: (b, 0, 0)),
        pl.BlockSpec((_R, _N, 1), lambda b: (b, 0, 0)),
        full((_N, 1)),
    ]
    out_shape = [
        jax.ShapeDtypeStruct((_N, _N, 1), _F32),
        jax.ShapeDtypeStruct((_N, _N, 1), _F32),
        jax.ShapeDtypeStruct((_N, 1), _F32),
    ]
    pred3, eff3, pri = pl.pallas_call(
        _dense_kernel,
        grid=(_NB,),
        in_specs=in_specs,
        out_specs=out_specs,
        out_shape=out_shape,
        scratch_shapes=[pltpu.VMEM((_N, _N), _F32), pltpu.VMEM((_N, _N), _F32)],
    )(emb, px, py, px, py, W1aT, W1bT, wd_row, g_row, W2T, b2r, W3T,
      b3r, We1T, be1r, We2T, be2r, Wp1T, bp1r, Wp2T, bp2r)
    return pred3.reshape(_N, _N), eff3.reshape(_N, _N), pri


import functools
from jax.experimental.pallas import tpu_sc as plsc


def _ranks(pri_row, pri_col, ca_row, ca_col, n0_row, n0_col):
    iota_sub = lax.broadcasted_iota(_I32, (_N, _N), 0).astype(_F32)
    iota_lan = lax.broadcasted_iota(_I32, (_N, _N), 1).astype(_F32)
    one = jnp.float32(1.0)
    zero = jnp.float32(0.0)
    isdef_r = n0_row < 0.0
    isdef_c = n0_col < 0.0
    issur_r = n0_row > 0.0
    issur_c = n0_col > 0.0
    beforeA = (ca_row == ca_col) & (
        (pri_row > pri_col) | ((pri_row == pri_col) & (iota_lan < iota_sub)))
    beforeB = (ca_col == ca_row) & (
        (pri_col > pri_row) | ((pri_col == pri_row) & (iota_sub < iota_lan)))
    rd_col = jnp.sum(jnp.where(beforeA & isdef_r, one, zero), axis=1,
                     keepdims=True)
    rs_col = jnp.sum(jnp.where(beforeA & issur_r, one, zero), axis=1,
                     keepdims=True)
    rd_row = jnp.sum(jnp.where(beforeB & isdef_c, one, zero), axis=0,
                     keepdims=True)
    rs_row = jnp.sum(jnp.where(beforeB & issur_c, one, zero), axis=0,
                     keepdims=True)
    return (iota_sub, iota_lan, isdef_r, isdef_c, issur_r, issur_c,
            rd_col, rs_col, rd_row, rs_row)


def _pack_kernel(pred_ref, eff_ref, pri_row_ref, pri_col_ref,
                 ca_row_ref, ca_col_ref, n0_row_ref, n0_col_ref,
                 predp_ref, effp_ref, need0_ref, avail0_ref, meta_ref):
    pred = pred_ref[...]
    eff = eff_ref[...]
    prir, pric = pri_row_ref[...], pri_col_ref[...]
    car, cac = ca_row_ref[...], ca_col_ref[...]
    n0r, n0c = n0_row_ref[...], n0_col_ref[...]
    (iota_sub, iota_lan, isdef_r, isdef_c, issur_r, issur_c,
     rd_col, rs_col, rd_row, rs_row) = _ranks(prir, pric, car, cac, n0r, n0c)
    one = jnp.float32(1.0)
    zero = jnp.float32(0.0)

    Gd = jnp.where(isdef_c & (rd_col == iota_lan), one, zero)
    Gs = jnp.where(issur_c & (rs_col == iota_lan), one, zero)

    for c in range(_NCLUST):
        fc = jnp.float32(c)
        ScT = jnp.where(issur_r & (rs_row == iota_sub) & (car == fc),
                        one, zero)
        Dc = jnp.where(isdef_c & (rd_col == iota_lan) & (cac == fc),
                       one, zero)
        predp_ref[c] = _mm(_mm(ScT, pred, _HI), Dc, _HI)
        effp_ref[c] = _mm(_mm(ScT, eff, _HI), Dc, _HI)

    io8 = lax.broadcasted_iota(_I32, (_NCLUST, _N), 0).astype(_F32)
    CAoh = jnp.where(car == io8, one, zero)
    need0_ref[...] = _mm(CAoh, Gd * (-n0c), _HI)
    avail0_ref[...] = _mm(CAoh, Gs * n0c, _HI)
    scnt8 = jnp.sum(CAoh * jnp.where(issur_r, one, zero), axis=1,
                    keepdims=True)
    dcnt8 = jnp.sum(CAoh * jnp.where(isdef_r, one, zero), axis=1,
                    keepdims=True)
    iol8 = lax.broadcasted_iota(_I32, (_NCLUST, _N), 1)
    meta_ref[...] = jnp.where(iol8 == 0, scnt8, zero) \
        + jnp.where(iol8 == 1, dcnt8, zero)


def _run_pack(pred, eff, pri_row, pri_col, ca_row, ca_col, n0_row, n0_col):
    full = lambda shp: pl.BlockSpec(shp, lambda: tuple(0 for _ in shp))
    args = (pred, eff, pri_row, pri_col, ca_row, ca_col, n0_row, n0_col)
    out_shape = [
        jax.ShapeDtypeStruct((_NCLUST, _N, _N), _F32),
        jax.ShapeDtypeStruct((_NCLUST, _N, _N), _F32),
        jax.ShapeDtypeStruct((_NCLUST, _N), _F32),
        jax.ShapeDtypeStruct((_NCLUST, _N), _F32),
        jax.ShapeDtypeStruct((_NCLUST, _N), _F32),
    ]
    return pl.pallas_call(
        _pack_kernel,
        in_specs=[full(a.shape) for a in args],
        out_specs=[full(s.shape) for s in out_shape],
        out_shape=out_shape,
    )(*args)


def _sc_greedy(predp, effp, need0, avail0, meta8):
    mesh = plsc.VectorSubcoreMesh(core_axis_name="c", subcore_axis_name="s")

    @functools.partial(
        pl.kernel, mesh=mesh,
        out_type=[
            jax.ShapeDtypeStruct((_NCLUST, _N, _N), _F32),
            jax.ShapeDtypeStruct((_NCLUST, _N, _N), _F32),
            jax.ShapeDtypeStruct((_NCLUST, _N), _F32),
        ],
        scratch_types=[
            pltpu.VMEM((_N, _N), _F32),
            pltpu.VMEM((_N, _N), _F32),
            pltpu.VMEM((_N, _N), _F32),
            pltpu.VMEM((_N, _N), _F32),
            pltpu.VMEM((_N,), _F32),
            pltpu.VMEM((2 * _N,), _F32),
            pltpu.VMEM((_N,), _F32),
        ],
    )
    def k(predp_hbm, effp_hbm, need0_hbm, avail0_hbm, meta_hbm,
          shp_hbm, edp_hbm, needp_hbm,
          pred_vm, eff_vm, sh_vm, ed_vm, needed_vm, avail_vm, meta_vm):
        core = lax.axis_index("c")
        tid = lax.axis_index("s")
        iota16 = lax.iota(_I32, 16)
        z16 = jnp.zeros((16,), _F32)

        def cumsum16(x):
            for kk in (1, 2, 4, 8):
                sh = jnp.take(x, jnp.maximum(iota16 - kk, 0))
                x = x + jnp.where(iota16 >= kk, sh, 0.0)
            return x

        @pl.when((core == 0) & (tid < _NCLUST))
        def _():
            pltpu.sync_copy(predp_hbm.at[tid], pred_vm)
            pltpu.sync_copy(effp_hbm.at[tid], eff_vm)
            pltpu.sync_copy(need0_hbm.at[tid], needed_vm)
            pltpu.sync_copy(avail0_hbm.at[tid], avail_vm.at[pl.ds(0, _N)])
            pltpu.sync_copy(meta_hbm.at[tid], meta_vm)
            mv = meta_vm[pl.ds(0, 16)]
            scnt = mv[0].astype(_I32)
            dcnt = mv[1].astype(_I32)

            def zrow(r, carry):
                for q in range(8):
                    sh_vm[r, pl.ds(q * 16, 16)] = z16
                    ed_vm[r, pl.ds(q * 16, 16)] = z16
                return carry

            lax.fori_loop(0, _N, zrow, 0)

            def srow(i, carry):
                avail = avail_vm[pl.ds(i, 16)][0]
                for q in range(8):
                    lanes = (iota16 + q * 16) < dcnt
                    p16 = pred_vm[i, pl.ds(q * 16, 16)]
                    e16 = eff_vm[i, pl.ds(q * 16, 16)]
                    needed16 = needed_vm[pl.ds(q * 16, 16)]
                    gate0 = lanes & (needed16 > 0.0)
                    cap = jnp.where(gate0, jnp.minimum(needed16, p16), 0.0)
                    cinc = cumsum16(cap)
                    cex = cinc - cap
                    upd = gate0 & (avail - cex > 0.0)
                    act = jnp.where(
                        upd,
                        jnp.minimum(avail, cinc) - jnp.minimum(avail, cex),
                        0.0)
                    dlv = act * e16
                    needed_vm[pl.ds(q * 16, 16)] = \
                        needed16 - jnp.where(upd, dlv, 0.0)
                    sh_vm[i, pl.ds(q * 16, 16)] = act
                    ed_vm[i, pl.ds(q * 16, 16)] = \
                        jnp.where(upd, e16 - 1.0, 0.0)
                    avail = jnp.maximum(avail - cinc[15], 0.0)
                return carry

            lax.fori_loop(0, scnt, srow, 0)

            pltpu.sync_copy(sh_vm, shp_hbm.at[tid])
            pltpu.sync_copy(ed_vm, edp_hbm.at[tid])
            pltpu.sync_copy(needed_vm, needp_hbm.at[tid])

    return k(predp, effp, need0, avail0, meta8)


def _epi_kernel(shp_ref, edp_ref, needp_ref, pri_row_ref, pri_col_ref,
                ca_row_ref, ca_col_ref, n0_row_ref, n0_col_ref,
                sharing_ref, effmat_ref, net_ref, esent_ref, erecv_ref,
                total_ref):
    prir, pric = pri_row_ref[...], pri_col_ref[...]
    car, cac = ca_row_ref[...], ca_col_ref[...]
    n0r, n0c = n0_row_ref[...], n0_col_ref[...]
    (iota_sub, iota_lan, isdef_r, isdef_c, issur_r, issur_c,
     rd_col, rs_col, rd_row, rs_row) = _ranks(prir, pric, car, cac, n0r, n0c)
    one = jnp.float32(1.0)
    zero = jnp.float32(0.0)

    sharing = jnp.zeros((_N, _N), _F32)
    effd = jnp.zeros((_N, _N), _F32)
    neededU = jnp.zeros((1, _N), _F32)
    for c in range(_NCLUST):
        fc = jnp.float32(c)
        Sc = jnp.where(issur_c & (rs_col == iota_lan) & (cac == fc),
                       one, zero)
        DcT = jnp.where(isdef_r & (rd_row == iota_sub) & (car == fc),
                        one, zero)
        sharing = sharing + _mm(_mm(Sc, shp_ref[c], _HI), DcT, _HI)
        effd = effd + _mm(_mm(Sc, edp_ref[c], _HI), DcT, _HI)
        neededU = neededU + _mm(needp_ref[c:c + 1, :], DcT, _HI)

    effmat = 1.0 + effd
    net_out = jnp.where(isdef_r, -neededU, n0r)
    sharing_ref[...] = sharing
    effmat_ref[...] = effmat
    net_ref[...] = net_out
    esent_ref[...] = jnp.sum(sharing, axis=1, keepdims=True)
    erecv_ref[...] = jnp.sum(sharing * effmat, axis=0, keepdims=True)
    total_ref[...] = jnp.sum(sharing).reshape(1, 1)


def _run_epi(shp, edp, needp, pri_row, pri_col, ca_row, ca_col,
             n0_row, n0_col):
    full = lambda shp_: pl.BlockSpec(shp_, lambda: tuple(0 for _ in shp_))
    args = (shp, edp, needp, pri_row, pri_col, ca_row, ca_col, n0_row, n0_col)
    out_shape = [
        jax.ShapeDtypeStruct((_N, _N), _F32),
        jax.ShapeDtypeStruct((_N, _N), _F32),
        jax.ShapeDtypeStruct((1, _N), _F32),
        jax.ShapeDtypeStruct((_N, 1), _F32),
        jax.ShapeDtypeStruct((1, _N), _F32),
        jax.ShapeDtypeStruct((1, 1), _F32),
    ]
    return pl.pallas_call(
        _epi_kernel,
        in_specs=[full(a.shape) for a in args],
        out_specs=[full(s.shape) for s in out_shape],
        out_shape=out_shape,
    )(*args)


def kernel(embeddings, cluster_assignments, generation, consumption,
           positions, current_hour, W1, b1, W2, b2, W3, b3,
           We1, be1, We2, be2, Wp1, bp1, Wp2, bp2):
    emb = embeddings[0].astype(_F32)
    pos = positions[0].astype(_F32)
    px = pos[:, 0:1]
    py = pos[:, 1:2]
    ca_row = cluster_assignments[0].astype(_F32).reshape(1, _N)
    ca_col = ca_row.reshape(_N, 1)
    n0_row = (generation - consumption).astype(_F32).reshape(1, _N)
    n0_col = n0_row.reshape(_N, 1)
    hour = jnp.asarray(current_hour / 24.0, _F32).reshape(1, 1)

    W1aT = W1[:, :_N].T
    W1bT = W1[:, _N:2 * _N].T
    wd_row = W1[:, 2 * _N:2 * _N + 1].reshape(1, _N)
    wh_row = W1[:, 2 * _N + 1:2 * _N + 2].reshape(1, _N)
    g_row = hour * wh_row + b1.reshape(1, -1)

    pred, eff, pri_col = _run_dense(
        emb, px, py, W1aT, W1bT, wd_row, g_row,
        W2.T, b2.reshape(1, -1), W3.T, b3.reshape(1, -1),
        We1.T, be1.reshape(1, -1), We2.T, be2.reshape(1, -1),
        Wp1.T, bp1.reshape(1, -1), Wp2.T, bp2.reshape(1, -1))

    pri_row = pri_col.reshape(1, _N)

    predp, effp, need0, avail0, meta8 = _run_pack(
        pred, eff, pri_row, pri_col, ca_row, ca_col, n0_row, n0_col)

    shp, edp, needp = _sc_greedy(predp, effp, need0, avail0, meta8)

    sharing, effmat, net, esent, erecv, total = _run_epi(
        shp, edp, needp, pri_row, pri_col, ca_row, ca_col, n0_row, n0_col)

    return (sharing.reshape(1, _N, _N), effmat.reshape(1, _N, _N),
            total.reshape(()), esent.reshape(1, _N),
            erecv.reshape(1, _N), net.reshape(1, _N))


# speedup vs baseline: 1.0857x; 1.0857x over previous
import jax
import jax.numpy as jnp
from jax import lax
from jax.experimental import pallas as pl
from jax.experimental.pallas import tpu as pltpu

_N = 128
_R = 16
_NB = _N // _R
_NCLUST = 8
_F32 = jnp.float32
_I32 = jnp.int32

_HI = lax.Precision.HIGHEST


def _mm(a, b, precision=None):
    return lax.dot_general(a, b, dimension_numbers=(((1,), (0,)), ((), ())),
                           preferred_element_type=_F32, precision=precision)


def _dense_kernel(emb_ref, px_ref, py_ref, pxb_ref, pyb_ref,
                  W1aT_ref, W1bT_ref, wd_ref, g_ref,
                  W2T_ref, b2_ref, W3T_ref, b3_ref,
                  We1T_ref, be1_ref, We2T_ref, be2_ref,
                  Wp1T_ref, bp1_ref, Wp2T_ref, bp2_ref,
                  pred_ref, eff_ref, pri_ref, A_ref, B_ref):
    b = pl.program_id(0)

    @pl.when(b == 0)
    def _():
        A_ref[...] = _mm(emb_ref[...], W1aT_ref[...]) + g_ref[...]
        B_ref[...] = _mm(emb_ref[...], W1bT_ref[...])
        hp = jnp.maximum(_mm(emb_ref[...], Wp1T_ref[...]) + bp1_ref[...], 0.0)
        pri_ref[...] = jax.nn.sigmoid(_mm(hp, Wp2T_ref[...]) + bp2_ref[...])

    pxi = pxb_ref[...].reshape(_R, 1, 1)
    pyi = pyb_ref[...].reshape(_R, 1, 1)
    pxj = px_ref[...].reshape(1, _N, 1)
    pyj = py_ref[...].reshape(1, _N, 1)
    dx = jnp.broadcast_to(pxi, (_R, _N, 1)) - jnp.broadcast_to(pxj, (_R, _N, 1))
    dy = jnp.broadcast_to(pyi, (_R, _N, 1)) - jnp.broadcast_to(pyj, (_R, _N, 1))
    dist = jnp.sqrt(dx * dx + dy * dy).reshape(_R * _N, 1)

    he = jnp.maximum(_mm(dist * (1.0 / 1000.0), We1T_ref[...]) + be1_ref[...],
                     0.0)
    se = jax.nn.sigmoid(_mm(he, We2T_ref[...]) + be2_ref[...])
    eff_ref[...] = (0.85 + 0.13 * se).reshape(_R, _N, 1)

    a3 = A_ref[pl.ds(b * _R, _R), :].reshape(_R, 1, _N)
    ab = jnp.broadcast_to(a3, (_R, _N, _N)).reshape(_R * _N, _N)
    bb = jnp.broadcast_to(B_ref[...].reshape(1, _N, _N),
                          (_R, _N, _N)).reshape(_R * _N, _N)
    h1 = jnp.maximum(ab + bb + dist * wd_ref[...], 0.0)
    h2 = jnp.maximum(_mm(h1, W2T_ref[...]) + b2_ref[...], 0.0)
    pr = _mm(h2, W3T_ref[...]) + b3_ref[...]
    pred_ref[...] = jax.nn.softplus(pr).reshape(_R, _N, 1)


def _run_dense(emb, px, py, W1aT, W1bT, wd_row, g_row, W2T, b2r,
               W3T, b3r, We1T, be1r, We2T, be2r, Wp1T, bp1r, Wp2T, bp2r):
    full = lambda shp: pl.BlockSpec(shp, lambda b: tuple(0 for _ in shp))
    in_specs = [
        full((_N, _N)),
        full((_N, 1)), full((_N, 1)),
        pl.BlockSpec((_R, 1), lambda b: (b, 0)),
        pl.BlockSpec((_R, 1), lambda b: (b, 0)),
        full(W1aT.shape), full(W1bT.shape), full(wd_row.shape),
        full(g_row.shape), full(W2T.shape), full(b2r.shape),
        full(W3T.shape), full(b3r.shape), full(We1T.shape), full(be1r.shape),
        full(We2T.shape), full(be2r.shape), full(Wp1T.shape), full(bp1r.shape),
        full(Wp2T.shape), full(bp2r.shape),
    ]
    out_specs = [
        pl.BlockSpec((_R, _N, 1), lambda b: (b, 0, 0)),
        pl.BlockSpec((_R, _N, 1), lambda b: (b, 0, 0)),
        full((_N, 1)),
    ]
    out_shape = [
        jax.ShapeDtypeStruct((_N, _N, 1), _F32),
        jax.ShapeDtypeStruct((_N, _N, 1), _F32),
        jax.ShapeDtypeStruct((_N, 1), _F32),
    ]
    pred3, eff3, pri = pl.pallas_call(
        _dense_kernel,
        grid=(_NB,),
        in_specs=in_specs,
        out_specs=out_specs,
        out_shape=out_shape,
        scratch_shapes=[pltpu.VMEM((_N, _N), _F32), pltpu.VMEM((_N, _N), _F32)],
    )(emb, px, py, px, py, W1aT, W1bT, wd_row, g_row, W2T, b2r, W3T,
      b3r, We1T, be1r, We2T, be2r, Wp1T, bp1r, Wp2T, bp2r)
    return pred3.reshape(_N, _N), eff3.reshape(_N, _N), pri


import functools
from jax.experimental.pallas import tpu_sc as plsc


def _granks(pri_row, pri_col, ca_row, ca_col, n0_row, n0_col):
    iota_sub = lax.broadcasted_iota(_I32, (_N, _N), 0).astype(_F32)
    iota_lan = lax.broadcasted_iota(_I32, (_N, _N), 1).astype(_F32)
    one = jnp.float32(1.0)
    zero = jnp.float32(0.0)
    isdef_r = n0_row < 0.0
    isdef_c = n0_col < 0.0
    issur_r = n0_row > 0.0
    issur_c = n0_col > 0.0
    beforeA = (ca_row < ca_col) | ((ca_row == ca_col) & (
        (pri_row > pri_col) | ((pri_row == pri_col) & (iota_lan < iota_sub))))
    beforeB = (ca_col < ca_row) | ((ca_col == ca_row) & (
        (pri_col > pri_row) | ((pri_col == pri_row) & (iota_sub < iota_lan))))
    rd_col = jnp.sum(jnp.where(beforeA & isdef_r, one, zero), axis=1,
                     keepdims=True)
    rs_col = jnp.sum(jnp.where(beforeA & issur_r, one, zero), axis=1,
                     keepdims=True)
    rd_row = jnp.sum(jnp.where(beforeB & isdef_c, one, zero), axis=0,
                     keepdims=True)
    rs_row = jnp.sum(jnp.where(beforeB & issur_c, one, zero), axis=0,
                     keepdims=True)
    D2 = jnp.where(isdef_c & (rd_col == iota_lan), one, zero)
    D2T = jnp.where(isdef_r & (rd_row == iota_sub), one, zero)
    S2 = jnp.where(issur_c & (rs_col == iota_lan), one, zero)
    S2T = jnp.where(issur_r & (rs_row == iota_sub), one, zero)
    return iota_sub, iota_lan, isdef_r, issur_r, D2, D2T, S2, S2T


def _pack_kernel(pred_ref, eff_ref, pri_row_ref, pri_col_ref,
                 ca_row_ref, ca_col_ref, n0_row_ref, n0_col_ref,
                 predp_ref, effp_ref, need0_ref, avail0_ref, segm_ref,
                 meta_ref):
    pred = pred_ref[...]
    eff = eff_ref[...]
    prir, pric = pri_row_ref[...], pri_col_ref[...]
    car, cac = ca_row_ref[...], ca_col_ref[...]
    n0r, n0c = n0_row_ref[...], n0_col_ref[...]
    (iota_sub, iota_lan, isdef_r, issur_r,
     D2, D2T, S2, S2T) = _granks(prir, pric, car, cac, n0r, n0c)
    one = jnp.float32(1.0)
    zero = jnp.float32(0.0)

    predp_ref[...] = _mm(_mm(S2T, pred, _HI), D2, _HI)
    effp_ref[...] = _mm(_mm(S2T, eff, _HI), D2, _HI)
    need0_ref[...] = -_mm(n0r, D2, _HI)
    avail0_ref[...] = _mm(n0r, S2, _HI)

    segr = _mm(car, D2, _HI)
    io8 = lax.broadcasted_iota(_I32, (_NCLUST, _N), 0).astype(_F32)
    segm_ref[...] = jnp.where(segr == io8, one, zero)
    cnt8 = jnp.sum(jnp.where((car == io8) & issur_r, one, zero),
                   axis=1, keepdims=True)
    start8 = jnp.sum(jnp.where((car < io8) & issur_r, one, zero),
                     axis=1, keepdims=True)
    iol8 = lax.broadcasted_iota(_I32, (_NCLUST, _N), 1)
    meta_ref[...] = jnp.where(iol8 == 0, start8, zero) \
        + jnp.where(iol8 == 1, cnt8, zero)


def _run_pack(pred, eff, pri_row, pri_col, ca_row, ca_col, n0_row, n0_col):
    full = lambda shp: pl.BlockSpec(shp, lambda: tuple(0 for _ in shp))
    args = (pred, eff, pri_row, pri_col, ca_row, ca_col, n0_row, n0_col)
    out_shape = [
        jax.ShapeDtypeStruct((_N, _N), _F32),
        jax.ShapeDtypeStruct((_N, _N), _F32),
        jax.ShapeDtypeStruct((1, _N), _F32),
        jax.ShapeDtypeStruct((1, _N), _F32),
        jax.ShapeDtypeStruct((_NCLUST, _N), _F32),
        jax.ShapeDtypeStruct((_NCLUST, _N), _F32),
    ]
    return pl.pallas_call(
        _pack_kernel,
        in_specs=[full(a.shape) for a in args],
        out_specs=[full(s.shape) for s in out_shape],
        out_shape=out_shape,
    )(*args)


def _sc_greedy(predp, effp, need0, avail0, segm, meta8):
    mesh = plsc.VectorSubcoreMesh(core_axis_name="c", subcore_axis_name="s")

    @functools.partial(
        pl.kernel, mesh=mesh,
        out_type=[
            jax.ShapeDtypeStruct((_N, _N), _F32),
            jax.ShapeDtypeStruct((_N, _N), _F32),
            jax.ShapeDtypeStruct((_NCLUST, _N), _F32),
        ],
        scratch_types=[
            pltpu.VMEM((_N, _N), _F32),
            pltpu.VMEM((_N, _N), _F32),
            pltpu.VMEM((_N,), _F32),
            pltpu.VMEM((2 * _N,), _F32),
            pltpu.VMEM((_N,), _F32),
            pltpu.VMEM((_N,), _F32),
            pltpu.VMEM((_N,), _F32),
            pltpu.VMEM((_N,), _F32),
        ],
    )
    def k(predp_hbm, effp_hbm, need0_hbm, avail0_hbm, segm_hbm, meta_hbm,
          shp_hbm, edp_hbm, needp_hbm,
          pred_vm, eff_vm, needed_vm, availp_vm, mask_vm, meta_vm,
          rowbuf, edbuf):
        core = lax.axis_index("c")
        tid = lax.axis_index("s")
        iota16 = lax.iota(_I32, 16)

        def cumsum16(x):
            for kk in (1, 2, 4, 8):
                sh = jnp.take(x, jnp.maximum(iota16 - kk, 0))
                x = x + jnp.where(iota16 >= kk, sh, 0.0)
            return x

        @pl.when((core == 0) & (tid < _NCLUST))
        def _():
            pltpu.sync_copy(predp_hbm, pred_vm)
            pltpu.sync_copy(effp_hbm, eff_vm)
            pltpu.sync_copy(need0_hbm, needed_vm)
            pltpu.sync_copy(avail0_hbm, availp_vm.at[pl.ds(0, _N)])
            pltpu.sync_copy(segm_hbm.at[tid], mask_vm)
            pltpu.sync_copy(meta_hbm.at[tid], meta_vm)
            mv = meta_vm[pl.ds(0, 16)]
            sstart = mv[0].astype(_I32)
            scnt = mv[1].astype(_I32)

            def srow(i, carry):
                r = sstart + i
                avail = availp_vm[pl.ds(r, 16)][0]
                for q in range(8):
                    sl = pl.ds(q * 16, 16)
                    lanes = mask_vm[sl] > 0.5
                    p16 = pred_vm[r, sl]
                    e16 = eff_vm[r, sl]
                    needed16 = needed_vm[sl]
                    gate0 = lanes & (needed16 > 0.0)
                    cap = jnp.where(gate0, jnp.minimum(needed16, p16), 0.0)
                    cinc = cumsum16(cap)
                    cex = cinc - cap
                    upd = gate0 & (avail - cex > 0.0)
                    act = jnp.where(
                        upd,
                        jnp.minimum(avail, cinc) - jnp.minimum(avail, cex),
                        0.0)
                    dlv = act * e16
                    needed_vm[sl] = needed16 - jnp.where(upd, dlv, 0.0)
                    rowbuf[sl] = act
                    edbuf[sl] = jnp.where(upd, e16 - 1.0, 0.0)
                    avail = jnp.maximum(avail - cinc[15], 0.0)
                pltpu.sync_copy(rowbuf, shp_hbm.at[r])
                pltpu.sync_copy(edbuf, edp_hbm.at[r])
                return carry

            lax.fori_loop(0, scnt, srow, 0)

            for q in range(8):
                sl = pl.ds(q * 16, 16)
                rowbuf[sl] = needed_vm[sl] * mask_vm[sl]
            pltpu.sync_copy(rowbuf, needp_hbm.at[tid])

    return k(predp, effp, need0, avail0, segm, meta8)


def _epi_kernel(shp_ref, edp_ref, needp_ref, pri_row_ref, pri_col_ref,
                ca_row_ref, ca_col_ref, n0_row_ref, n0_col_ref,
                sharing_ref, effmat_ref, net_ref, esent_ref, erecv_ref,
                total_ref):
    prir, pric = pri_row_ref[...], pri_col_ref[...]
    car, cac = ca_row_ref[...], ca_col_ref[...]
    n0r, n0c = n0_row_ref[...], n0_col_ref[...]
    (iota_sub, iota_lan, isdef_r, issur_r,
     D2, D2T, S2, S2T) = _granks(prir, pric, car, cac, n0r, n0c)
    one = jnp.float32(1.0)
    zero = jnp.float32(0.0)

    nsur = jnp.sum(jnp.where(issur_r, one, zero), axis=1, keepdims=True)
    valid = iota_sub < nsur
    shp = jnp.where(valid, shp_ref[...], zero)
    edp = jnp.where(valid, edp_ref[...], zero)

    sharing = _mm(_mm(S2, shp, _HI), D2T, _HI)
    effmat = 1.0 + _mm(_mm(S2, edp, _HI), D2T, _HI)
    needslots = jnp.sum(needp_ref[...], axis=0, keepdims=True)
    neededU = _mm(needslots, D2T, _HI)
    net_out = jnp.where(isdef_r, -neededU, n0r)

    sharing_ref[...] = sharing
    effmat_ref[...] = effmat
    net_ref[...] = net_out
    esent_ref[...] = jnp.sum(sharing, axis=1, keepdims=True)
    erecv_ref[...] = jnp.sum(sharing * effmat, axis=0, keepdims=True)
    total_ref[...] = jnp.sum(sharing).reshape(1, 1)


def _run_epi(shp, edp, needp, pri_row, pri_col, ca_row, ca_col,
             n0_row, n0_col):
    full = lambda shp_: pl.BlockSpec(shp_, lambda: tuple(0 for _ in shp_))
    args = (shp, edp, needp, pri_row, pri_col, ca_row, ca_col, n0_row, n0_col)
    out_shape = [
        jax.ShapeDtypeStruct((_N, _N), _F32),
        jax.ShapeDtypeStruct((_N, _N), _F32),
        jax.ShapeDtypeStruct((1, _N), _F32),
        jax.ShapeDtypeStruct((_N, 1), _F32),
        jax.ShapeDtypeStruct((1, _N), _F32),
        jax.ShapeDtypeStruct((1, 1), _F32),
    ]
    return pl.pallas_call(
        _epi_kernel,
        in_specs=[full(a.shape) for a in args],
        out_specs=[full(s.shape) for s in out_shape],
        out_shape=out_shape,
    )(*args)


def kernel(embeddings, cluster_assignments, generation, consumption,
           positions, current_hour, W1, b1, W2, b2, W3, b3,
           We1, be1, We2, be2, Wp1, bp1, Wp2, bp2):
    emb = embeddings[0].astype(_F32)
    pos = positions[0].astype(_F32)
    px = pos[:, 0:1]
    py = pos[:, 1:2]
    ca_row = cluster_assignments[0].astype(_F32).reshape(1, _N)
    ca_col = ca_row.reshape(_N, 1)
    n0_row = (generation - consumption).astype(_F32).reshape(1, _N)
    n0_col = n0_row.reshape(_N, 1)
    hour = jnp.asarray(current_hour / 24.0, _F32).reshape(1, 1)

    W1aT = W1[:, :_N].T
    W1bT = W1[:, _N:2 * _N].T
    wd_row = W1[:, 2 * _N:2 * _N + 1].reshape(1, _N)
    wh_row = W1[:, 2 * _N + 1:2 * _N + 2].reshape(1, _N)
    g_row = hour * wh_row + b1.reshape(1, -1)

    pred, eff, pri_col = _run_dense(
        emb, px, py, W1aT, W1bT, wd_row, g_row,
        W2.T, b2.reshape(1, -1), W3.T, b3.reshape(1, -1),
        We1.T, be1.reshape(1, -1), We2.T, be2.reshape(1, -1),
        Wp1.T, bp1.reshape(1, -1), Wp2.T, bp2.reshape(1, -1))

    pri_row = pri_col.reshape(1, _N)

    predp, effp, need0, avail0, segm, meta8 = _run_pack(
        pred, eff, pri_row, pri_col, ca_row, ca_col, n0_row, n0_col)

    shp, edp, needp = _sc_greedy(predp, effp, need0.reshape(_N),
                                 avail0.reshape(_N), segm, meta8)

    sharing, effmat, net, esent, erecv, total = _run_epi(
        shp, edp, needp, pri_row, pri_col, ca_row, ca_col, n0_row, n0_col)

    return (sharing.reshape(1, _N, _N), effmat.reshape(1, _N, _N),
            total.reshape(()), esent.reshape(1, _N),
            erecv.reshape(1, _N), net.reshape(1, _N))
